# baseline (speedup 1.0000x reference)
import functools

import jax
import jax.numpy as jnp
from jax import lax
from jax.experimental import pallas as pl
from jax.experimental.pallas import tpu as pltpu
from jax.experimental.pallas import tpu_sc as plsc

EMB = 64
N = 16384 * 200
NC, NS = 2, 16
NW = NC * NS
PER_W = N // NW
CHUNK = 512
NCHUNK = PER_W // CHUNK

_MESH = plsc.VectorSubcoreMesh(core_axis_name="c", subcore_axis_name="s")


@functools.partial(
    pl.kernel,
    out_type=jax.ShapeDtypeStruct((N, EMB), jnp.float32),
    mesh=_MESH,
    scratch_types=[
        pltpu.VMEM((CHUNK,), jnp.int32),
        pltpu.VMEM((CHUNK, EMB), jnp.float32),
        pltpu.SemaphoreType.DMA,
    ],
    compiler_params=pltpu.CompilerParams(use_tc_tiling_on_sc=False),
)
def _emb_lookup(x_hbm, tab_hbm, out_hbm, idx_v, rows_v, sem):
    wid = lax.axis_index("s") * NC + lax.axis_index("c")
    base = wid * PER_W

    def body(g, carry):
        off = base + g * CHUNK
        pltpu.sync_copy(x_hbm.at[pl.ds(off, CHUNK)], idx_v)
        pltpu.async_copy(tab_hbm.at[idx_v], rows_v, sem).wait()
        pltpu.sync_copy(rows_v, out_hbm.at[pl.ds(off, CHUNK)])
        return carry

    lax.fori_loop(0, NCHUNK, body, 0)


def kernel(x, connectivity_embedding):
    x1d = x.reshape(-1)
    out = _emb_lookup(x1d, connectivity_embedding)
    return out.reshape(x.shape + (EMB,))


# speedup vs baseline: 2.9631x; 2.9631x over previous
import functools

import jax
import jax.numpy as jnp
from jax import lax
from jax.experimental import pallas as pl
from jax.experimental.pallas import tpu as pltpu
from jax.experimental.pallas import tpu_sc as plsc

EMB = 64
N = 16384 * 200
NC, NS = 2, 16
NW = NC * NS
PER_W = N // NW
CHUNK = 512
NCHUNK = PER_W // CHUNK
NPAIR = NCHUNK // 2
GROUPS = CHUNK // 16
CW = CHUNK * EMB

_MESH = plsc.VectorSubcoreMesh(core_axis_name="c", subcore_axis_name="s")


@functools.partial(
    pl.kernel,
    out_type=jax.ShapeDtypeStruct((N * EMB,), jnp.float32),
    mesh=_MESH,
    scratch_types=[
        pltpu.VMEM((5 * EMB,), jnp.float32),
        pltpu.VMEM((CHUNK,), jnp.int32),
        pltpu.VMEM((CHUNK,), jnp.int32),
        pltpu.VMEM((CW,), jnp.float32),
        pltpu.VMEM((CW,), jnp.float32),
        pltpu.SemaphoreType.DMA,
        pltpu.SemaphoreType.DMA,
    ],
    compiler_params=pltpu.CompilerParams(
        use_tc_tiling_on_sc=False, needs_layout_passes=False),
)
def _emb_lookup(x_hbm, tab_hbm, out_hbm, tab_v, idx_a, idx_b, rows_a, rows_b,
                sem_a, sem_b):
    wid = lax.axis_index("s") * NC + lax.axis_index("c")
    base = wid * PER_W

    pltpu.sync_copy(tab_hbm, tab_v)
    lane = lax.iota(jnp.int32, 16)
    lane64 = lane * EMB

    def build(idx_v, rows_v):
        def group(k, carry):
            idxv = idx_v[pl.ds(k * 16, 16)]
            addr = idxv * EMB
            pos = lane64 + k * (16 * EMB)
            for c in range(EMB):
                v = plsc.load_gather(tab_v, [addr + c])
                plsc.store_scatter(rows_v, [pos + c], v)
            return carry
        lax.fori_loop(0, GROUPS, group, 0, unroll=False)

    def pair(t, carry):
        off0 = base + (2 * t) * CHUNK

        pltpu.sync_copy(x_hbm.at[pl.ds(off0, CHUNK)], idx_a)

        @pl.when(t > 0)
        def _():
            pltpu.make_async_copy(
                rows_a, out_hbm.at[pl.ds((off0 - 2 * CHUNK) * EMB, CW)],
                sem_a).wait()

        build(idx_a, rows_a)
        pltpu.make_async_copy(
            rows_a, out_hbm.at[pl.ds(off0 * EMB, CW)], sem_a).start()

        off1 = off0 + CHUNK
        pltpu.sync_copy(x_hbm.at[pl.ds(off1, CHUNK)], idx_b)

        @pl.when(t > 0)
        def _():
            pltpu.make_async_copy(
                rows_b, out_hbm.at[pl.ds((off1 - 2 * CHUNK) * EMB, CW)],
                sem_b).wait()

        build(idx_b, rows_b)
        pltpu.make_async_copy(
            rows_b, out_hbm.at[pl.ds(off1 * EMB, CW)], sem_b).start()
        return carry

    lax.fori_loop(0, NPAIR, pair, 0)

    last0 = base + (NCHUNK - 2) * CHUNK
    pltpu.make_async_copy(
        rows_a, out_hbm.at[pl.ds(last0 * EMB, CW)], sem_a).wait()
    pltpu.make_async_copy(
        rows_b, out_hbm.at[pl.ds((last0 + CHUNK) * EMB, CW)], sem_b).wait()


def kernel(x, connectivity_embedding):
    x1d = x.reshape(-1)
    tab1d = connectivity_embedding.reshape(-1)
    out = _emb_lookup(x1d, tab1d)
    return out.reshape(x.shape + (EMB,))


# speedup vs baseline: 3.4022x; 1.1482x over previous
import functools

import jax
import jax.numpy as jnp
from jax import lax
from jax.experimental import pallas as pl
from jax.experimental.pallas import tpu as pltpu
from jax.experimental.pallas import tpu_sc as plsc

EMB = 64
N = 16384 * 200
NC, NS = 2, 16
NW = NC * NS
PER_W = N // NW
CHUNK = 512
NCHUNK = PER_W // CHUNK
NPAIR = NCHUNK // 2
GROUPS = CHUNK // 16
CW = CHUNK * EMB

_MESH = plsc.VectorSubcoreMesh(core_axis_name="c", subcore_axis_name="s")


@functools.partial(
    pl.kernel,
    out_type=jax.ShapeDtypeStruct((N * EMB,), jnp.float32),
    mesh=_MESH,
    scratch_types=[
        pltpu.VMEM((5 * EMB,), jnp.float32),
        pltpu.VMEM((CHUNK,), jnp.int32),
        pltpu.VMEM((CHUNK,), jnp.int32),
        pltpu.VMEM((CW,), jnp.float32),
        pltpu.VMEM((CW,), jnp.float32),
        pltpu.SemaphoreType.DMA,
        pltpu.SemaphoreType.DMA,
    ],
    compiler_params=pltpu.CompilerParams(
        use_tc_tiling_on_sc=False, needs_layout_passes=False),
)
def _emb_lookup(x_hbm, tab_hbm, out_hbm, tab_v, idx_a, idx_b, rows_a, rows_b,
                sem_a, sem_b):
    wid = lax.axis_index("s") * NC + lax.axis_index("c")
    base = wid * PER_W

    pltpu.sync_copy(tab_hbm, tab_v)
    lane = lax.iota(jnp.int32, 16)
    lane64 = lane * EMB

    def build(idx_v, rows_v):
        @plsc.parallel_loop(0, GROUPS, step=1, unroll=2)
        def group(k):
            idxv = idx_v[pl.ds(k * 16, 16)]
            addr = idxv * EMB
            pos = lane64 + k * (16 * EMB)
            for c in range(EMB):
                v = plsc.load_gather(tab_v, [addr + c])
                plsc.store_scatter(rows_v, [pos + c], v)

    def pair(t, carry):
        off0 = base + (2 * t) * CHUNK

        pltpu.sync_copy(x_hbm.at[pl.ds(off0, CHUNK)], idx_a)

        @pl.when(t > 0)
        def _():
            pltpu.make_async_copy(
                rows_a, out_hbm.at[pl.ds((off0 - 2 * CHUNK) * EMB, CW)],
                sem_a).wait()

        build(idx_a, rows_a)
        pltpu.make_async_copy(
            rows_a, out_hbm.at[pl.ds(off0 * EMB, CW)], sem_a).start()

        off1 = off0 + CHUNK
        pltpu.sync_copy(x_hbm.at[pl.ds(off1, CHUNK)], idx_b)

        @pl.when(t > 0)
        def _():
            pltpu.make_async_copy(
                rows_b, out_hbm.at[pl.ds((off1 - 2 * CHUNK) * EMB, CW)],
                sem_b).wait()

        build(idx_b, rows_b)
        pltpu.make_async_copy(
            rows_b, out_hbm.at[pl.ds(off1 * EMB, CW)], sem_b).start()
        return carry

    lax.fori_loop(0, NPAIR, pair, 0)

    last0 = base + (NCHUNK - 2) * CHUNK
    pltpu.make_async_copy(
        rows_a, out_hbm.at[pl.ds(last0 * EMB, CW)], sem_a).wait()
    pltpu.make_async_copy(
        rows_b, out_hbm.at[pl.ds((last0 + CHUNK) * EMB, CW)], sem_b).wait()


def kernel(x, connectivity_embedding):
    x1d = x.reshape(-1)
    tab1d = connectivity_embedding.reshape(-1)
    out = _emb_lookup(x1d, tab1d)
    return out.reshape(x.shape + (EMB,))


# speedup vs baseline: 11.5411x; 3.3922x over previous
import functools

import jax
import jax.numpy as jnp
from jax import lax
from jax.experimental import pallas as pl
from jax.experimental.pallas import tpu as pltpu
from jax.experimental.pallas import tpu_sc as plsc

EMB = 64
N = 16384 * 200
NC, NS = 2, 16
NW = NC * NS
PER_W = N // NW
CHUNK = 512
NCHUNK = PER_W // CHUNK
NPAIR = NCHUNK // 2
GROUPS = CHUNK // 16
CW = CHUNK * EMB

_MESH = plsc.VectorSubcoreMesh(core_axis_name="c", subcore_axis_name="s")


@functools.partial(
    pl.kernel,
    out_type=jax.ShapeDtypeStruct((N * EMB,), jnp.float32),
    mesh=_MESH,
    scratch_types=[
        pltpu.VMEM((5 * EMB,), jnp.float32),
        pltpu.VMEM((CHUNK,), jnp.int32),
        pltpu.VMEM((CHUNK,), jnp.int32),
        pltpu.VMEM((CW,), jnp.float32),
        pltpu.VMEM((CW,), jnp.float32),
        pltpu.SemaphoreType.DMA,
        pltpu.SemaphoreType.DMA,
    ],
    compiler_params=pltpu.CompilerParams(
        use_tc_tiling_on_sc=False, needs_layout_passes=False),
)
def _emb_lookup(x_hbm, tab_hbm, out_hbm, tab_v, idx_a, idx_b, rows_a, rows_b,
                sem_a, sem_b):
    wid = lax.axis_index("s") * NC + lax.axis_index("c")
    base = wid * PER_W

    pltpu.sync_copy(tab_hbm, tab_v)
    lane = lax.iota(jnp.int32, 16)
    lane64 = lane * EMB

    def build(idx_v, rows_v):
        @plsc.parallel_loop(0, GROUPS, step=1, unroll=1)
        def group(k):
            idxv = idx_v[pl.ds(k * 16, 16)]
            for r in range(16):
                tbase = idxv[r] * EMB
                obase = (k * 16 + r) * EMB
                for j in range(EMB // 16):
                    v = tab_v[pl.ds(tbase + 16 * j, 16)]
                    rows_v[pl.ds(obase + 16 * j, 16)] = v

    def pair(t, carry):
        off0 = base + (2 * t) * CHUNK

        pltpu.sync_copy(x_hbm.at[pl.ds(off0, CHUNK)], idx_a)

        @pl.when(t > 0)
        def _():
            pltpu.make_async_copy(
                rows_a, out_hbm.at[pl.ds((off0 - 2 * CHUNK) * EMB, CW)],
                sem_a).wait()

        build(idx_a, rows_a)
        pltpu.make_async_copy(
            rows_a, out_hbm.at[pl.ds(off0 * EMB, CW)], sem_a).start()

        off1 = off0 + CHUNK
        pltpu.sync_copy(x_hbm.at[pl.ds(off1, CHUNK)], idx_b)

        @pl.when(t > 0)
        def _():
            pltpu.make_async_copy(
                rows_b, out_hbm.at[pl.ds((off1 - 2 * CHUNK) * EMB, CW)],
                sem_b).wait()

        build(idx_b, rows_b)
        pltpu.make_async_copy(
            rows_b, out_hbm.at[pl.ds(off1 * EMB, CW)], sem_b).start()
        return carry

    lax.fori_loop(0, NPAIR, pair, 0)

    last0 = base + (NCHUNK - 2) * CHUNK
    pltpu.make_async_copy(
        rows_a, out_hbm.at[pl.ds(last0 * EMB, CW)], sem_a).wait()
    pltpu.make_async_copy(
        rows_b, out_hbm.at[pl.ds((last0 + CHUNK) * EMB, CW)], sem_b).wait()


def kernel(x, connectivity_embedding):
    x1d = x.reshape(-1)
    tab1d = connectivity_embedding.reshape(-1)
    out = _emb_lookup(x1d, tab1d)
    return out.reshape(x.shape + (EMB,))


# speedup vs baseline: 11.9384x; 1.0344x over previous
import functools

import jax
import jax.numpy as jnp
from jax import lax
from jax.experimental import pallas as pl
from jax.experimental.pallas import tpu as pltpu
from jax.experimental.pallas import tpu_sc as plsc

EMB = 64
N = 16384 * 200
NC, NS = 2, 16
NW = NC * NS
PER_W = N // NW
CHUNK = 800
NCHUNK = PER_W // CHUNK
NPAIR = NCHUNK // 2
GROUPS = CHUNK // 16
CW = CHUNK * EMB

_MESH = plsc.VectorSubcoreMesh(core_axis_name="c", subcore_axis_name="s")


@functools.partial(
    pl.kernel,
    out_type=jax.ShapeDtypeStruct((N * EMB,), jnp.float32),
    mesh=_MESH,
    scratch_types=[
        pltpu.VMEM((5 * EMB,), jnp.float32),
        pltpu.VMEM((CHUNK,), jnp.int32),
        pltpu.VMEM((CHUNK,), jnp.int32),
        pltpu.VMEM((CW,), jnp.float32),
        pltpu.VMEM((CW,), jnp.float32),
        pltpu.SemaphoreType.DMA,
        pltpu.SemaphoreType.DMA,
    ],
    compiler_params=pltpu.CompilerParams(
        use_tc_tiling_on_sc=False, needs_layout_passes=False),
)
def _emb_lookup(x_hbm, tab_hbm, out_hbm, tab_v, idx_a, idx_b, rows_a, rows_b,
                sem_a, sem_b):
    wid = lax.axis_index("s") * NC + lax.axis_index("c")
    base = wid * PER_W

    pltpu.sync_copy(tab_hbm, tab_v)
    lane = lax.iota(jnp.int32, 16)
    lane64 = lane * EMB

    def build(idx_v, rows_v):
        @plsc.parallel_loop(0, GROUPS, step=1, unroll=1)
        def group(k):
            idxv = idx_v[pl.ds(k * 16, 16)]
            for r in range(16):
                tbase = idxv[r] * EMB
                obase = (k * 16 + r) * EMB
                for j in range(EMB // 16):
                    v = tab_v[pl.ds(tbase + 16 * j, 16)]
                    rows_v[pl.ds(obase + 16 * j, 16)] = v

    def pair(t, carry):
        off0 = base + (2 * t) * CHUNK

        pltpu.sync_copy(x_hbm.at[pl.ds(off0, CHUNK)], idx_a)

        @pl.when(t > 0)
        def _():
            pltpu.make_async_copy(
                rows_a, out_hbm.at[pl.ds((off0 - 2 * CHUNK) * EMB, CW)],
                sem_a).wait()

        build(idx_a, rows_a)
        pltpu.make_async_copy(
            rows_a, out_hbm.at[pl.ds(off0 * EMB, CW)], sem_a).start()

        off1 = off0 + CHUNK
        pltpu.sync_copy(x_hbm.at[pl.ds(off1, CHUNK)], idx_b)

        @pl.when(t > 0)
        def _():
            pltpu.make_async_copy(
                rows_b, out_hbm.at[pl.ds((off1 - 2 * CHUNK) * EMB, CW)],
                sem_b).wait()

        build(idx_b, rows_b)
        pltpu.make_async_copy(
            rows_b, out_hbm.at[pl.ds(off1 * EMB, CW)], sem_b).start()
        return carry

    lax.fori_loop(0, NPAIR, pair, 0)

    last0 = base + (NCHUNK - 2) * CHUNK
    pltpu.make_async_copy(
        rows_a, out_hbm.at[pl.ds(last0 * EMB, CW)], sem_a).wait()
    pltpu.make_async_copy(
        rows_b, out_hbm.at[pl.ds((last0 + CHUNK) * EMB, CW)], sem_b).wait()


def kernel(x, connectivity_embedding):
    x1d = x.reshape(-1)
    tab1d = connectivity_embedding.reshape(-1)
    out = _emb_lookup(x1d, tab1d)
    return out.reshape(x.shape + (EMB,))
